# Initial kernel scaffold; baseline (speedup 1.0000x reference)
import jax
import jax.numpy as jnp
from jax.experimental import pallas as pl


def kernel(inputs, embedding):
    raise NotImplementedError("write your pallas kernel here")


# speedup vs baseline: 4.0886x; 4.0886x over previous
import functools

import jax
import jax.numpy as jnp
from jax import lax
from jax.experimental import pallas as pl
from jax.experimental.pallas import tpu as pltpu
from jax.experimental.pallas import tpu_sc as plsc

_NUM_CORES = 2
_NUM_SUBCORES = 16
_NUM_WORKERS = _NUM_CORES * _NUM_SUBCORES

_CHUNK = 128
_GROUP = 4


@functools.lru_cache(maxsize=None)
def _make_gather(n_rows: int, d: int):
    n_per_w = n_rows // _NUM_WORKERS
    chunks_per_w = n_per_w // _CHUNK
    groups_per_w = chunks_per_w // _GROUP
    rows_per_group = _CHUNK * _GROUP
    assert n_rows % (_NUM_WORKERS * rows_per_group) == 0

    mesh = plsc.VectorSubcoreMesh(
        core_axis_name="c", subcore_axis_name="s",
        num_cores=_NUM_CORES, num_subcores=_NUM_SUBCORES)

    @functools.partial(
        pl.kernel,
        out_type=jax.ShapeDtypeStruct((n_rows, d), jnp.float32),
        mesh=mesh,
        scratch_types=[
            pltpu.VMEM((chunks_per_w, _CHUNK), jnp.int32),
            pltpu.VMEM((rows_per_group, d), jnp.float32),
            pltpu.SemaphoreType.DMA,
        ],
        compiler_params=pltpu.CompilerParams(use_tc_tiling_on_sc=False),
    )
    def gather_kernel(table, idx_hbm, out_hbm, idx_v, buf, sem):
        wid = lax.axis_index("s") * _NUM_CORES + lax.axis_index("c")
        idx_row_base = wid * chunks_per_w
        out_base = wid * n_per_w
        pltpu.sync_copy(idx_hbm.at[pl.ds(idx_row_base, chunks_per_w)], idx_v)

        @pl.loop(0, groups_per_w)
        def _(g):
            copies = [
                pltpu.async_copy(
                    table.at[idx_v.at[g * _GROUP + j]],
                    buf.at[pl.ds(j * _CHUNK, _CHUNK)],
                    sem)
                for j in range(_GROUP)
            ]
            for c in copies:
                c.wait()
            pltpu.sync_copy(
                buf, out_hbm.at[pl.ds(out_base + g * rows_per_group,
                                      rows_per_group)])

    return gather_kernel


def kernel(inputs, embedding):
    b, s = inputs.shape
    v, d = embedding.shape
    n_rows = b * s
    idx = inputs.reshape(n_rows // _CHUNK, _CHUNK).astype(jnp.int32)
    out = _make_gather(n_rows, d)(embedding, idx)
    return out.reshape(b, s, d)


# speedup vs baseline: 4.2672x; 1.0437x over previous
import functools

import jax
import jax.numpy as jnp
from jax import lax
from jax.experimental import pallas as pl
from jax.experimental.pallas import tpu as pltpu
from jax.experimental.pallas import tpu_sc as plsc

_NUM_CORES = 2
_NUM_SUBCORES = 16
_NUM_WORKERS = _NUM_CORES * _NUM_SUBCORES

_CHUNK = 128
_GROUP = 2
_NBUF = 4


@functools.lru_cache(maxsize=None)
def _make_gather(n_rows: int, d: int):
    n_per_w = n_rows // _NUM_WORKERS
    chunks_per_w = n_per_w // _CHUNK
    groups_per_w = chunks_per_w // _GROUP
    rows_per_group = _CHUNK * _GROUP
    assert n_rows % (_NUM_WORKERS * rows_per_group) == 0
    assert groups_per_w % _NBUF == 0 and groups_per_w >= 2 * _NBUF

    mesh = plsc.VectorSubcoreMesh(
        core_axis_name="c", subcore_axis_name="s",
        num_cores=_NUM_CORES, num_subcores=_NUM_SUBCORES)

    @functools.partial(
        pl.kernel,
        out_type=jax.ShapeDtypeStruct((n_rows, d), jnp.float32),
        mesh=mesh,
        scratch_types=[
            pltpu.VMEM((chunks_per_w, _CHUNK), jnp.int32),
            [pltpu.VMEM((rows_per_group, d), jnp.float32)] * _NBUF,
            [pltpu.SemaphoreType.DMA] * _NBUF,
            [pltpu.SemaphoreType.DMA] * _NBUF,
        ],
        compiler_params=pltpu.CompilerParams(use_tc_tiling_on_sc=False),
    )
    def gather_kernel(table, idx_hbm, out_hbm, idx_v, bufs, gsems, ssems):
        wid = lax.axis_index("s") * _NUM_CORES + lax.axis_index("c")
        idx_row_base = wid * chunks_per_w
        out_base = wid * n_per_w
        pltpu.sync_copy(idx_hbm.at[pl.ds(idx_row_base, chunks_per_w)], idx_v)

        def fire_gathers(g, b):
            for j in range(_GROUP):
                pltpu.async_copy(
                    table.at[idx_v.at[g * _GROUP + j]],
                    bufs[b].at[pl.ds(j * _CHUNK, _CHUNK)],
                    gsems[b])

        def wait_gathers(b):
            for j in range(_GROUP):
                pltpu.make_async_copy(
                    table.at[idx_v.at[j]],
                    bufs[b].at[pl.ds(j * _CHUNK, _CHUNK)],
                    gsems[b]).wait()

        def fire_store(g, b):
            pltpu.async_copy(
                bufs[b],
                out_hbm.at[pl.ds(out_base + g * rows_per_group,
                                 rows_per_group)],
                ssems[b])

        def wait_store(b):
            pltpu.make_async_copy(
                bufs[b], out_hbm.at[pl.ds(out_base, rows_per_group)],
                ssems[b]).wait()

        for g in range(_NBUF):
            fire_gathers(g, g)
            if g >= 1:
                wait_gathers(g - 1)
                fire_store(g - 1, g - 1)

        @pl.loop(1, groups_per_w // _NBUF)
        def _(t):
            for b in range(_NBUF):
                g = _NBUF * t + b
                wait_store(b)
                fire_gathers(g, b)
                pb = (b - 1) % _NBUF
                wait_gathers(pb)
                fire_store(g - 1, pb)

        last = groups_per_w - 1
        lb = last % _NBUF
        wait_gathers(lb)
        fire_store(last, lb)
        for b in range(_NBUF):
            wait_store(b)

    return gather_kernel


def kernel(inputs, embedding):
    b, s = inputs.shape
    v, d = embedding.shape
    n_rows = b * s
    idx = inputs.reshape(n_rows // _CHUNK, _CHUNK).astype(jnp.int32)
    out = _make_gather(n_rows, d)(embedding, idx)
    return out.reshape(b, s, d)
